# Initial kernel scaffold; baseline (speedup 1.0000x reference)
import jax
import jax.numpy as jnp
from jax.experimental import pallas as pl


def kernel(query, ref_points, Wv, bv, Ws, bso, Wa, ba, Wo, bo):
    raise NotImplementedError("write your pallas kernel here")


# speedup vs baseline: 7.8450x; 7.8450x over previous
import functools

import jax
import jax.numpy as jnp
from jax import lax
from jax.experimental import pallas as pl
from jax.experimental.pallas import tpu as pltpu
from jax.experimental.pallas import tpu_sc as plsc

EMBED = 256
HH = 129
WW = 256
NH = 8
NP = 4
DH = EMBED // NH
LANES = NH * NP * 4
TAPS = NP * 4

T = 256

NC = 2
NS = 16
NW = NC * NS
G = 64


def _k1_body(q_ref, refx_ref, refy_ref, wv_ref, bv_ref, wsx_ref, bsox_ref,
             wsy_ref, bsoy_ref, wa_ref, ba_ref, ssum_ref, eexp_ref,
             val_ref, idx_ref, wgt_ref, *, nq):
    b = pl.program_id(0)
    q = q_ref[0]
    hi = jax.lax.Precision.HIGHEST

    val_ref[0] = (jnp.dot(q, wv_ref[...], precision=hi,
                          preferred_element_type=jnp.float32) + bv_ref[...])

    offx = jnp.dot(q, wsx_ref[...], precision=hi,
                   preferred_element_type=jnp.float32) + bsox_ref[...]
    offy = jnp.dot(q, wsy_ref[...], precision=hi,
                   preferred_element_type=jnp.float32) + bsoy_ref[...]

    logits = jnp.dot(q, wa_ref[...], precision=hi,
                     preferred_element_type=jnp.float32) + ba_ref[...]
    m = jnp.max(logits, axis=-1, keepdims=True)
    e = jnp.exp(logits - m)
    denom = jnp.dot(e, ssum_ref[...], precision=hi,
                    preferred_element_type=jnp.float32)
    aw128 = jnp.dot(e / denom, eexp_ref[...], precision=hi,
                    preferred_element_type=jnp.float32)

    lane = lax.broadcasted_iota(jnp.int32, (T, LANES), 1)
    h = lane >> 4
    tx = (lane & 1).astype(jnp.float32)
    ty = ((lane >> 1) & 1).astype(jnp.float32)

    gx = refx_ref[...] + offx
    gy = refy_ref[...] + offy
    x0 = jnp.floor(gx)
    y0 = jnp.floor(gy)
    fx = gx - x0
    fy = gy - y0
    xi = x0 + tx
    yi = y0 + ty
    wx = jnp.where(tx > 0.5, fx, 1.0 - fx)
    wy = jnp.where(ty > 0.5, fy, 1.0 - fy)
    valid = ((xi >= 0.0) & (xi <= WW - 1) & (yi >= 0.0) & (yi <= HH - 1))
    xc = jnp.clip(xi, 0.0, WW - 1).astype(jnp.int32)
    yc = jnp.clip(yi, 0.0, HH - 1).astype(jnp.int32)
    qsrc = yc * WW + xc
    row = b * nq + qsrc
    idx_ref[0] = (row << 3) + h
    wgt_ref[0] = jnp.where(valid, aw128 * wx * wy, 0.0)


def _k1(query, refx, refy, Wv, bv, Wsx, bsox, Wsy, bsoy, Wa, ba, Ssum, Eexp):
    B, Nq, D = query.shape
    grid = (B, Nq // T)
    full = lambda shape: pl.BlockSpec(shape, lambda b, j: (0,) * len(shape))
    return pl.pallas_call(
        functools.partial(_k1_body, nq=Nq),
        grid=grid,
        in_specs=[
            pl.BlockSpec((1, T, D), lambda b, j: (b, j, 0)),
            pl.BlockSpec((T, 1), lambda b, j: (j, 0)),
            pl.BlockSpec((T, 1), lambda b, j: (j, 0)),
            full((D, D)), full((1, D)),
            full((D, LANES)), full((1, LANES)),
            full((D, LANES)), full((1, LANES)),
            full((D, NH * NP)), full((1, NH * NP)),
            full((NH * NP, NH * NP)), full((NH * NP, LANES)),
        ],
        out_specs=[
            pl.BlockSpec((1, T, D), lambda b, j: (b, j, 0)),
            pl.BlockSpec((1, T, LANES), lambda b, j: (b, j, 0)),
            pl.BlockSpec((1, T, LANES), lambda b, j: (b, j, 0)),
        ],
        out_shape=[
            jax.ShapeDtypeStruct((B, Nq, D), jnp.float32),
            jax.ShapeDtypeStruct((B, Nq, LANES), jnp.int32),
            jax.ShapeDtypeStruct((B, Nq, LANES), jnp.float32),
        ],
    )(query, refx, refy, Wv, bv, Wsx, bsox, Wsy, bsoy, Wa, ba, Ssum, Eexp)


def _k2_body(table, idxh, wgth, outh, idx_v, wgt_v, rows_v, out_v, sem,
             *, npw, nchunk):
    wid = lax.axis_index("s") * NC + lax.axis_index("c")

    def chunk_body(c, carry):
        ibase = pl.multiple_of(wid * npw + c * G, G)
        ebase = pl.multiple_of(ibase * TAPS, G * TAPS)
        rbase = pl.multiple_of(ebase // 128, (G * TAPS) // 128)
        pltpu.sync_copy(idxh.at[pl.ds(rbase, (G * TAPS) // 128)], idx_v)
        pltpu.sync_copy(wgth.at[pl.ds(ebase, G * TAPS)], wgt_v)
        copies = [
            pltpu.async_copy(table.at[idx_v.at[j]],
                             rows_v.at[pl.ds(j * 128, 128)], sem)
            for j in range((G * TAPS) // 128)
        ]
        for cp in copies:
            cp.wait()

        def item_body(i, carry2):
            e0 = i * TAPS
            acc0 = jnp.zeros((16,), jnp.float32)
            acc1 = jnp.zeros((16,), jnp.float32)
            for k in range(TAPS):
                wk = plsc.load_gather(
                    wgt_v, [jnp.broadcast_to(e0 + k, (16,)).astype(jnp.int32)])
                acc0 = acc0 + wk * rows_v[e0 + k, pl.ds(0, 16)]
                acc1 = acc1 + wk * rows_v[e0 + k, pl.ds(16, 16)]
            out_v[i, pl.ds(0, 16)] = acc0
            out_v[i, pl.ds(16, 16)] = acc1
            return carry2

        lax.fori_loop(0, G, item_body, 0, unroll=False)
        pltpu.sync_copy(out_v, outh.at[pl.ds(ibase, G)])
        return carry

    lax.fori_loop(0, nchunk, chunk_body, 0, unroll=False)


def _k2(table, idx2d, wgt_flat, nitems):
    npw = nitems // NW
    nchunk = npw // G
    mesh = plsc.VectorSubcoreMesh(core_axis_name="c", subcore_axis_name="s",
                                  num_cores=NC, num_subcores=NS)
    kern = functools.partial(
        pl.kernel,
        mesh=mesh,
        out_type=jax.ShapeDtypeStruct((nitems, DH), jnp.float32),
        scratch_types=[
            pltpu.VMEM(((G * TAPS) // 128, 128), jnp.int32),
            pltpu.VMEM((G * TAPS,), jnp.float32),
            pltpu.VMEM((G * TAPS, DH), jnp.float32),
            pltpu.VMEM((G, DH), jnp.float32),
            pltpu.SemaphoreType.DMA,
        ],
        compiler_params=pltpu.CompilerParams(needs_layout_passes=False,
                                             use_tc_tiling_on_sc=False),
    )(functools.partial(_k2_body, npw=npw, nchunk=nchunk))
    return kern(table, idx2d, wgt_flat)


def _k3_body(s_ref, q_ref, wo_ref, bo_ref, out_ref):
    hi = jax.lax.Precision.HIGHEST
    out_ref[0] = (jnp.dot(s_ref[0], wo_ref[...], precision=hi,
                          preferred_element_type=jnp.float32)
                  + bo_ref[...] + 2.0 * q_ref[0])


def _k3(sampled, query, Wo, bo):
    B, Nq, D = query.shape
    grid = (B, Nq // T)
    return pl.pallas_call(
        _k3_body,
        grid=grid,
        in_specs=[
            pl.BlockSpec((1, T, D), lambda b, j: (b, j, 0)),
            pl.BlockSpec((1, T, D), lambda b, j: (b, j, 0)),
            pl.BlockSpec((D, D), lambda b, j: (0, 0)),
            pl.BlockSpec((1, D), lambda b, j: (0, 0)),
        ],
        out_specs=pl.BlockSpec((1, T, D), lambda b, j: (b, j, 0)),
        out_shape=jax.ShapeDtypeStruct((B, Nq, D), jnp.float32),
    )(sampled, query, Wo, bo)


def kernel(query, ref_points, Wv, bv, Ws, bso, Wa, ba, Wo, bo):
    B, Nq, D = query.shape
    nitems = B * Nq * NH

    Wsx = jnp.repeat(Ws[:, 0::2], 4, axis=1)
    Wsy = jnp.repeat(Ws[:, 1::2], 4, axis=1)
    bsox = jnp.repeat(bso[0::2], 4)[None, :]
    bsoy = jnp.repeat(bso[1::2], 4)[None, :]
    eye32 = jnp.eye(NH * NP, dtype=jnp.float32)
    Eexp = jnp.repeat(eye32, 4, axis=1)
    Ssum = jnp.repeat(jnp.repeat(jnp.eye(NH, dtype=jnp.float32), NP, axis=0),
                      NP, axis=1)
    refx = (ref_points[:, 0, 0] * WW - 0.5).reshape(Nq, 1)
    refy = (ref_points[:, 0, 1] * HH - 0.5).reshape(Nq, 1)

    value, idx, wgt = _k1(query, refx, refy, Wv, bv.reshape(1, -1),
                          Wsx, bsox, Wsy, bsoy, Wa, ba.reshape(1, -1),
                          Ssum, Eexp)

    table = value.reshape(B * Nq * NH, DH)
    idx2d = idx.reshape((nitems * TAPS) // 128, 128)
    wgt_flat = wgt.reshape(nitems * TAPS)
    sc_out = _k2(table, idx2d, wgt_flat, nitems)

    sampled = sc_out.reshape(B, Nq, D)
    return _k3(sampled, query, Wo, bo.reshape(1, -1))


# speedup vs baseline: 13.7034x; 1.7468x over previous
import functools

import jax
import jax.numpy as jnp
from jax import lax
from jax.experimental import pallas as pl
from jax.experimental.pallas import tpu as pltpu
from jax.experimental.pallas import tpu_sc as plsc

EMBED = 256
HH = 129
WW = 256
NH = 8
NP = 4
DH = EMBED // NH
LANES = NH * NP * 4
TAPS = NP * 4

T = 256

NC = 2
NS = 16
NW = NC * NS
G = 64


def _k1_body(q_ref, refx_ref, refy_ref, wv_ref, bv_ref, wsx_ref, bsox_ref,
             wsy_ref, bsoy_ref, wa_ref, ba_ref, ssum_ref, eexp_ref,
             val_ref, idx_ref, wgt_ref, *, nq):
    b = pl.program_id(0)
    q = q_ref[0]
    hi = None

    val_ref[0] = (jnp.dot(q, wv_ref[...], precision=hi,
                          preferred_element_type=jnp.float32) + bv_ref[...])

    offx = jnp.dot(q, wsx_ref[...], precision=hi,
                   preferred_element_type=jnp.float32) + bsox_ref[...]
    offy = jnp.dot(q, wsy_ref[...], precision=hi,
                   preferred_element_type=jnp.float32) + bsoy_ref[...]

    logits = jnp.dot(q, wa_ref[...], precision=hi,
                     preferred_element_type=jnp.float32) + ba_ref[...]
    m = jnp.max(logits, axis=-1, keepdims=True)
    e = jnp.exp(logits - m)
    denom = jnp.dot(e, ssum_ref[...], precision=hi,
                    preferred_element_type=jnp.float32)
    aw128 = jnp.dot(e / denom, eexp_ref[...], precision=hi,
                    preferred_element_type=jnp.float32)

    lane = lax.broadcasted_iota(jnp.int32, (T, LANES), 1)
    h = lane >> 4
    tx = (lane & 1).astype(jnp.float32)
    ty = ((lane >> 1) & 1).astype(jnp.float32)

    gx = refx_ref[...] + offx
    gy = refy_ref[...] + offy
    x0 = jnp.floor(gx)
    y0 = jnp.floor(gy)
    fx = gx - x0
    fy = gy - y0
    xi = x0 + tx
    yi = y0 + ty
    wx = jnp.where(tx > 0.5, fx, 1.0 - fx)
    wy = jnp.where(ty > 0.5, fy, 1.0 - fy)
    valid = ((xi >= 0.0) & (xi <= WW - 1) & (yi >= 0.0) & (yi <= HH - 1))
    xc = jnp.clip(xi, 0.0, WW - 1).astype(jnp.int32)
    yc = jnp.clip(yi, 0.0, HH - 1).astype(jnp.int32)
    qsrc = yc * WW + xc
    row = b * nq + qsrc
    idx_ref[0] = (row << 3) + h
    wgt_ref[0] = jnp.where(valid, aw128 * wx * wy, 0.0)


def _k1(query, refx, refy, Wv, bv, Wsx, bsox, Wsy, bsoy, Wa, ba, Ssum, Eexp):
    B, Nq, D = query.shape
    grid = (B, Nq // T)
    full = lambda shape: pl.BlockSpec(shape, lambda b, j: (0,) * len(shape))
    return pl.pallas_call(
        functools.partial(_k1_body, nq=Nq),
        grid=grid,
        in_specs=[
            pl.BlockSpec((1, T, D), lambda b, j: (b, j, 0)),
            pl.BlockSpec((T, 1), lambda b, j: (j, 0)),
            pl.BlockSpec((T, 1), lambda b, j: (j, 0)),
            full((D, D)), full((1, D)),
            full((D, LANES)), full((1, LANES)),
            full((D, LANES)), full((1, LANES)),
            full((D, NH * NP)), full((1, NH * NP)),
            full((NH * NP, NH * NP)), full((NH * NP, LANES)),
        ],
        out_specs=[
            pl.BlockSpec((1, T, D), lambda b, j: (b, j, 0)),
            pl.BlockSpec((1, T, LANES), lambda b, j: (b, j, 0)),
            pl.BlockSpec((1, T, LANES), lambda b, j: (b, j, 0)),
        ],
        out_shape=[
            jax.ShapeDtypeStruct((B, Nq, D), jnp.float32),
            jax.ShapeDtypeStruct((B, Nq, LANES), jnp.int32),
            jax.ShapeDtypeStruct((B, Nq, LANES), jnp.float32),
        ],
    )(query, refx, refy, Wv, bv, Wsx, bsox, Wsy, bsoy, Wa, ba, Ssum, Eexp)


def _k2_body(table, idxh, wgth, outh, idx_v, wgt_v, rows_v, out_v,
             sem_r0, sem_r1, sem_i0, sem_i1, sem_w0, sem_w1, sem_o0, sem_o1,
             *, npw, nchunk):
    wid = lax.axis_index("s") * NC + lax.axis_index("c")
    sem_r = (sem_r0, sem_r1)
    sem_i = (sem_i0, sem_i1)
    sem_w = (sem_w0, sem_w1)
    sem_o = (sem_o0, sem_o1)
    NIR = (G * TAPS) // 128

    def ibase(c):
        return pl.multiple_of(wid * npw + c * G, G)

    def ebase(c):
        return pl.multiple_of(ibase(c) * TAPS, G * TAPS)

    def rbase(c):
        return pl.multiple_of(ebase(c) // 128, NIR)

    def idx_copy(c, buf):
        return pltpu.make_async_copy(idxh.at[pl.ds(rbase(c), NIR)],
                                     idx_v.at[buf], sem_i[buf])

    def wgt_copy(c, buf):
        return pltpu.make_async_copy(wgth.at[pl.ds(ebase(c), G * TAPS)],
                                     wgt_v.at[buf], sem_w[buf])

    def gather_copy(c, buf, j):
        return pltpu.make_async_copy(
            table.at[idx_v.at[buf].at[j]],
            rows_v.at[buf].at[pl.ds(j * 128, 128)], sem_r[buf])

    def out_copy(c, buf):
        return pltpu.make_async_copy(out_v.at[buf],
                                     outh.at[pl.ds(ibase(c), G)], sem_o[buf])

    idx_copy(0, 0).start()
    wgt_copy(0, 0).start()
    idx_copy(1, 1).start()
    wgt_copy(1, 1).start()
    idx_copy(0, 0).wait()
    for j in range(NIR):
        gather_copy(0, 0, j).start()

    def pair_body(i, carry):
        for b in (0, 1):
            c = i * 2 + b
            nb = 1 - b
            for j in range(NIR):
                gather_copy(c, b, j).wait()
            wgt_copy(c, b).wait()

            @pl.when(c + 1 < nchunk)
            def _():
                idx_copy(c + 1, nb).wait()
                for j in range(NIR):
                    gather_copy(c + 1, nb, j).start()

            @pl.when(c >= 2)
            def _():
                out_copy(c - 2, b).wait()

            def item_body(k, carry2):
                e0 = k * TAPS
                acc0 = jnp.zeros((16,), jnp.float32)
                acc1 = jnp.zeros((16,), jnp.float32)
                for t in range(TAPS):
                    wk = plsc.load_gather(
                        wgt_v.at[b],
                        [jnp.broadcast_to(e0 + t, (16,)).astype(jnp.int32)])
                    acc0 = acc0 + wk * rows_v[b, e0 + t, pl.ds(0, 16)]
                    acc1 = acc1 + wk * rows_v[b, e0 + t, pl.ds(16, 16)]
                out_v[b, k, pl.ds(0, 16)] = acc0
                out_v[b, k, pl.ds(16, 16)] = acc1
                return carry2

            lax.fori_loop(0, G, item_body, 0, unroll=False)
            out_copy(c, b).start()

            @pl.when(c + 2 < nchunk)
            def _():
                idx_copy(c + 2, b).start()
                wgt_copy(c + 2, b).start()
        return carry

    lax.fori_loop(0, nchunk // 2, pair_body, 0, unroll=False)
    out_copy(nchunk - 2, 0).wait()
    out_copy(nchunk - 1, 1).wait()


def _k2(table, idx2d, wgt_flat, nitems):
    npw = nitems // NW
    nchunk = npw // G
    mesh = plsc.VectorSubcoreMesh(core_axis_name="c", subcore_axis_name="s",
                                  num_cores=NC, num_subcores=NS)
    kern = functools.partial(
        pl.kernel,
        mesh=mesh,
        out_type=jax.ShapeDtypeStruct((nitems, DH), jnp.float32),
        scratch_types=[
            pltpu.VMEM((2, (G * TAPS) // 128, 128), jnp.int32),
            pltpu.VMEM((2, G * TAPS), jnp.float32),
            pltpu.VMEM((2, G * TAPS, DH), jnp.float32),
            pltpu.VMEM((2, G, DH), jnp.float32),
        ] + [pltpu.SemaphoreType.DMA] * 8,
        compiler_params=pltpu.CompilerParams(needs_layout_passes=False,
                                             use_tc_tiling_on_sc=False),
    )(functools.partial(_k2_body, npw=npw, nchunk=nchunk))
    return kern(table, idx2d, wgt_flat)


def _k3_body(s_ref, q_ref, wo_ref, bo_ref, out_ref):
    hi = None
    out_ref[0] = (jnp.dot(s_ref[0], wo_ref[...], precision=hi,
                          preferred_element_type=jnp.float32)
                  + bo_ref[...] + 2.0 * q_ref[0])


def _k3(sampled, query, Wo, bo):
    B, Nq, D = query.shape
    grid = (B, Nq // T)
    return pl.pallas_call(
        _k3_body,
        grid=grid,
        in_specs=[
            pl.BlockSpec((1, T, D), lambda b, j: (b, j, 0)),
            pl.BlockSpec((1, T, D), lambda b, j: (b, j, 0)),
            pl.BlockSpec((D, D), lambda b, j: (0, 0)),
            pl.BlockSpec((1, D), lambda b, j: (0, 0)),
        ],
        out_specs=pl.BlockSpec((1, T, D), lambda b, j: (b, j, 0)),
        out_shape=jax.ShapeDtypeStruct((B, Nq, D), jnp.float32),
    )(sampled, query, Wo, bo)


def kernel(query, ref_points, Wv, bv, Ws, bso, Wa, ba, Wo, bo):
    B, Nq, D = query.shape
    nitems = B * Nq * NH

    Wsx = jnp.repeat(Ws[:, 0::2], 4, axis=1)
    Wsy = jnp.repeat(Ws[:, 1::2], 4, axis=1)
    bsox = jnp.repeat(bso[0::2], 4)[None, :]
    bsoy = jnp.repeat(bso[1::2], 4)[None, :]
    eye32 = jnp.eye(NH * NP, dtype=jnp.float32)
    Eexp = jnp.repeat(eye32, 4, axis=1)
    Ssum = jnp.repeat(jnp.repeat(jnp.eye(NH, dtype=jnp.float32), NP, axis=0),
                      NP, axis=1)
    refx = (ref_points[:, 0, 0] * WW - 0.5).reshape(Nq, 1)
    refy = (ref_points[:, 0, 1] * HH - 0.5).reshape(Nq, 1)

    value, idx, wgt = _k1(query, refx, refy, Wv, bv.reshape(1, -1),
                          Wsx, bsox, Wsy, bsoy, Wa, ba.reshape(1, -1),
                          Ssum, Eexp)

    table = value.reshape(B * Nq * NH, DH)
    idx2d = idx.reshape((nitems * TAPS) // 128, 128)
    wgt_flat = wgt.reshape(nitems * TAPS)
    sc_out = _k2(table, idx2d, wgt_flat, nitems)

    sampled = sc_out.reshape(B, Nq, D)
    return _k3(sampled, query, Wo, bo.reshape(1, -1))


# speedup vs baseline: 14.4563x; 1.0549x over previous
import functools

import jax
import jax.numpy as jnp
from jax import lax
from jax.experimental import pallas as pl
from jax.experimental.pallas import tpu as pltpu
from jax.experimental.pallas import tpu_sc as plsc

EMBED = 256
HH = 129
WW = 256
NH = 8
NP = 4
DH = EMBED // NH
LANES = NH * NP * 4
TAPS = NP * 4

T = 256

NC = 2
NS = 16
NW = NC * NS
G = 64


def _round_bf16_bits(v):
    bits = lax.bitcast_convert_type(v, jnp.int32)
    return ((bits + 0x7FFF + ((bits >> 16) & 1)) >> 16) & 0xFFFF


def _k1_body(q_ref, refx_ref, refy_ref, wve_ref, wvo_ref, bv_ref,
             wsx_ref, bsox_ref, wsy_ref, bsoy_ref, wa_ref, ba_ref,
             ssum_ref, eexp_ref, val_ref, idx_ref, wgt_ref, *, nq):
    b = pl.program_id(0)
    qf = q_ref[0]
    q = qf.astype(jnp.bfloat16)
    hi = None

    ve = jnp.dot(q, wve_ref[...], precision=hi,
                 preferred_element_type=jnp.float32) + bv_ref[0:1]
    vo = jnp.dot(q, wvo_ref[...], precision=hi,
                 preferred_element_type=jnp.float32) + bv_ref[1:2]
    val_ref[0] = _round_bf16_bits(ve) | (_round_bf16_bits(vo) << 16)

    offx = jnp.dot(q, wsx_ref[...], precision=hi,
                   preferred_element_type=jnp.float32) + bsox_ref[...]
    offy = jnp.dot(q, wsy_ref[...], precision=hi,
                   preferred_element_type=jnp.float32) + bsoy_ref[...]

    logits = jnp.dot(q, wa_ref[...], precision=hi,
                     preferred_element_type=jnp.float32) + ba_ref[...]
    m = jnp.max(logits, axis=-1, keepdims=True)
    e = jnp.exp(logits - m)
    denom = jnp.dot(e, ssum_ref[...], precision=hi,
                    preferred_element_type=jnp.float32)
    aw128 = jnp.dot(e / denom, eexp_ref[...], precision=hi,
                    preferred_element_type=jnp.float32)

    lane = lax.broadcasted_iota(jnp.int32, (T, LANES), 1)
    h = lane >> 4
    tx = (lane & 1).astype(jnp.float32)
    ty = ((lane >> 1) & 1).astype(jnp.float32)

    gx = refx_ref[...] + offx
    gy = refy_ref[...] + offy
    x0 = jnp.floor(gx)
    y0 = jnp.floor(gy)
    fx = gx - x0
    fy = gy - y0
    xi = x0 + tx
    yi = y0 + ty
    wx = jnp.where(tx > 0.5, fx, 1.0 - fx)
    wy = jnp.where(ty > 0.5, fy, 1.0 - fy)
    valid = ((xi >= 0.0) & (xi <= WW - 1) & (yi >= 0.0) & (yi <= HH - 1))
    xc = jnp.clip(xi, 0.0, WW - 1).astype(jnp.int32)
    yc = jnp.clip(yi, 0.0, HH - 1).astype(jnp.int32)
    qsrc = yc * WW + xc
    row = b * nq + qsrc
    idx_ref[0] = (row << 3) + h
    wgt_ref[0] = jnp.where(valid, aw128 * wx * wy, 0.0)


def _k1(query, refx, refy, Wve, Wvo, bv2, Wsx, bsox, Wsy, bsoy, Wa, ba,
        Ssum, Eexp):
    B, Nq, D = query.shape
    grid = (B, Nq // T)
    full = lambda shape: pl.BlockSpec(shape, lambda b, j: (0,) * len(shape))
    return pl.pallas_call(
        functools.partial(_k1_body, nq=Nq),
        grid=grid,
        in_specs=[
            pl.BlockSpec((1, T, D), lambda b, j: (b, j, 0)),
            pl.BlockSpec((T, 1), lambda b, j: (j, 0)),
            pl.BlockSpec((T, 1), lambda b, j: (j, 0)),
            full((D, LANES)), full((D, LANES)), full((2, LANES)),
            full((D, LANES)), full((1, LANES)),
            full((D, LANES)), full((1, LANES)),
            full((D, NH * NP)), full((1, NH * NP)),
            full((NH * NP, NH * NP)), full((NH * NP, LANES)),
        ],
        out_specs=[
            pl.BlockSpec((1, T, LANES), lambda b, j: (b, j, 0)),
            pl.BlockSpec((1, T, LANES), lambda b, j: (b, j, 0)),
            pl.BlockSpec((1, T, LANES), lambda b, j: (b, j, 0)),
        ],
        out_shape=[
            jax.ShapeDtypeStruct((B, Nq, LANES), jnp.int32),
            jax.ShapeDtypeStruct((B, Nq, LANES), jnp.int32),
            jax.ShapeDtypeStruct((B, Nq, LANES), jnp.float32),
        ],
    )(query, refx, refy, Wve, Wvo, bv2, Wsx, bsox, Wsy, bsoy, Wa, ba,
      Ssum, Eexp)


def _k2_body(table, idxh, wgth, outh, idx_v, wgt_v, rows_v, out_v,
             sem_r0, sem_r1, sem_i0, sem_i1, sem_w0, sem_w1, sem_o0, sem_o1,
             *, npw, nchunk):
    wid = lax.axis_index("s") * NC + lax.axis_index("c")
    sem_r = (sem_r0, sem_r1)
    sem_i = (sem_i0, sem_i1)
    sem_w = (sem_w0, sem_w1)
    sem_o = (sem_o0, sem_o1)
    NIR = (G * TAPS) // 128

    def ibase(c):
        return pl.multiple_of(wid * npw + c * G, G)

    def ebase(c):
        return pl.multiple_of(ibase(c) * TAPS, G * TAPS)

    def rbase(c):
        return pl.multiple_of(ebase(c) // 128, NIR)

    def idx_copy(c, buf):
        return pltpu.make_async_copy(idxh.at[pl.ds(rbase(c), NIR)],
                                     idx_v.at[buf], sem_i[buf])

    def wgt_copy(c, buf):
        return pltpu.make_async_copy(wgth.at[pl.ds(ebase(c), G * TAPS)],
                                     wgt_v.at[buf], sem_w[buf])

    def gather_copy(c, buf, j):
        return pltpu.make_async_copy(
            table.at[idx_v.at[buf].at[j]],
            rows_v.at[buf].at[pl.ds(j * 128, 128)], sem_r[buf])

    def out_copy(c, buf):
        return pltpu.make_async_copy(out_v.at[buf],
                                     outh.at[pl.ds(ibase(c), G)], sem_o[buf])

    idx_copy(0, 0).start()
    wgt_copy(0, 0).start()
    idx_copy(1, 1).start()
    wgt_copy(1, 1).start()
    idx_copy(0, 0).wait()
    for j in range(NIR):
        gather_copy(0, 0, j).start()

    def pair_body(i, carry):
        for b in (0, 1):
            c = i * 2 + b
            nb = 1 - b
            for j in range(NIR):
                gather_copy(c, b, j).wait()
            wgt_copy(c, b).wait()

            @pl.when(c + 1 < nchunk)
            def _():
                idx_copy(c + 1, nb).wait()
                for j in range(NIR):
                    gather_copy(c + 1, nb, j).start()

            @pl.when(c >= 2)
            def _():
                out_copy(c - 2, b).wait()

            def item_body(k, carry2):
                e0 = k * TAPS
                acc0 = jnp.zeros((16,), jnp.float32)
                acc1 = jnp.zeros((16,), jnp.float32)
                for t in range(TAPS):
                    wk = plsc.load_gather(
                        wgt_v.at[b],
                        [jnp.broadcast_to(e0 + t, (16,)).astype(jnp.int32)])
                    word = rows_v[b, e0 + t, pl.ds(0, 16)]
                    even = plsc.bitcast(word << 16, jnp.float32)
                    odd = plsc.bitcast(word & jnp.int32(-65536), jnp.float32)
                    acc0 = acc0 + wk * even
                    acc1 = acc1 + wk * odd
                out_v[b, k, pl.ds(0, 16)] = acc0
                out_v[b, k, pl.ds(16, 16)] = acc1
                return carry2

            lax.fori_loop(0, G, item_body, 0, unroll=False)
            out_copy(c, b).start()

            @pl.when(c + 2 < nchunk)
            def _():
                idx_copy(c + 2, b).start()
                wgt_copy(c + 2, b).start()
        return carry

    lax.fori_loop(0, nchunk // 2, pair_body, 0, unroll=False)
    out_copy(nchunk - 2, 0).wait()
    out_copy(nchunk - 1, 1).wait()


def _k2(table, idx2d, wgt_flat, nitems):
    npw = nitems // NW
    nchunk = npw // G
    mesh = plsc.VectorSubcoreMesh(core_axis_name="c", subcore_axis_name="s",
                                  num_cores=NC, num_subcores=NS)
    kern = functools.partial(
        pl.kernel,
        mesh=mesh,
        out_type=jax.ShapeDtypeStruct((nitems, DH), jnp.float32),
        scratch_types=[
            pltpu.VMEM((2, (G * TAPS) // 128, 128), jnp.int32),
            pltpu.VMEM((2, G * TAPS), jnp.float32),
            pltpu.VMEM((2, G * TAPS, DH // 2), jnp.int32),
            pltpu.VMEM((2, G, DH), jnp.float32),
        ] + [pltpu.SemaphoreType.DMA] * 8,
        compiler_params=pltpu.CompilerParams(needs_layout_passes=False,
                                             use_tc_tiling_on_sc=False),
    )(functools.partial(_k2_body, npw=npw, nchunk=nchunk))
    return kern(table, idx2d, wgt_flat)


def _k3_body(s_ref, q_ref, wo_ref, bo_ref, out_ref):
    out_ref[0] = (jnp.dot(s_ref[0].astype(jnp.bfloat16), wo_ref[...],
                          preferred_element_type=jnp.float32)
                  + bo_ref[...] + 2.0 * q_ref[0])


def _k3(sampled, query, Wo, bo):
    B, Nq, D = query.shape
    grid = (B, Nq // T)
    return pl.pallas_call(
        _k3_body,
        grid=grid,
        in_specs=[
            pl.BlockSpec((1, T, D), lambda b, j: (b, j, 0)),
            pl.BlockSpec((1, T, D), lambda b, j: (b, j, 0)),
            pl.BlockSpec((D, D), lambda b, j: (0, 0)),
            pl.BlockSpec((1, D), lambda b, j: (0, 0)),
        ],
        out_specs=pl.BlockSpec((1, T, D), lambda b, j: (b, j, 0)),
        out_shape=jax.ShapeDtypeStruct((B, Nq, D), jnp.float32),
    )(sampled, query, Wo, bo)


def kernel(query, ref_points, Wv, bv, Ws, bso, Wa, ba, Wo, bo):
    B, Nq, D = query.shape
    nitems = B * Nq * NH

    bf16 = jnp.bfloat16
    Wsx = jnp.repeat(Ws[:, 0::2], 4, axis=1).astype(bf16)
    Wsy = jnp.repeat(Ws[:, 1::2], 4, axis=1).astype(bf16)
    bsox = jnp.repeat(bso[0::2], 4)[None, :]
    bsoy = jnp.repeat(bso[1::2], 4)[None, :]
    Wve = Wv[:, 0::2].astype(bf16)
    Wvo = Wv[:, 1::2].astype(bf16)
    bv2 = jnp.stack([bv[0::2], bv[1::2]])
    eye32 = jnp.eye(NH * NP, dtype=jnp.float32)
    Eexp = jnp.repeat(eye32, 4, axis=1)
    Ssum = jnp.repeat(jnp.repeat(jnp.eye(NH, dtype=jnp.float32), NP, axis=0),
                      NP, axis=1)
    refx = (ref_points[:, 0, 0] * WW - 0.5).reshape(Nq, 1)
    refy = (ref_points[:, 0, 1] * HH - 0.5).reshape(Nq, 1)
    ch = jnp.arange(D)
    hh, rr = ch // DH, ch % DH
    perm = hh * DH + jnp.where(rr < DH // 2, 2 * rr, 2 * (rr - DH // 2) + 1)
    Wo_perm = Wo[perm, :].astype(bf16)

    value, idx, wgt = _k1(query, refx, refy, Wve, Wvo, bv2,
                          Wsx, bsox, Wsy, bsoy, Wa.astype(bf16),
                          ba.reshape(1, -1), Ssum, Eexp)

    table = value.reshape(B * Nq * NH, DH // 2)
    idx2d = idx.reshape((nitems * TAPS) // 128, 128)
    wgt_flat = wgt.reshape(nitems * TAPS)
    sc_out = _k2(table, idx2d, wgt_flat, nitems)

    sampled = sc_out.reshape(B, Nq, D)
    return _k3(sampled, query, Wo_perm, bo.reshape(1, -1))


# speedup vs baseline: 15.6978x; 1.0859x over previous
import functools

import jax
import jax.numpy as jnp
from jax import lax
from jax.experimental import pallas as pl
from jax.experimental.pallas import tpu as pltpu
from jax.experimental.pallas import tpu_sc as plsc

EMBED = 256
HH = 129
WW = 256
NH = 8
NP = 4
DH = EMBED // NH
LANES = NH * NP * 4
TAPS = NP * 4

T = 256

NC = 2
NS = 16
NW = NC * NS
G = 64


def _round_bf16_bits(v):
    bits = lax.bitcast_convert_type(v, jnp.int32)
    return ((bits + 0x7FFF + ((bits >> 16) & 1)) >> 16) & 0xFFFF


def _k1_body(q_ref, refx_ref, refy_ref, wve_ref, wvo_ref, bv_ref,
             wsx_ref, bsox_ref, wsy_ref, bsoy_ref, wa_ref, ba_ref,
             ssum_ref, eexp_ref, val_ref, idx_ref, wgt_ref, *, nq):
    b = pl.program_id(0)
    qf = q_ref[0]
    q = qf.astype(jnp.bfloat16)
    hi = None

    ve = jnp.dot(q, wve_ref[...], precision=hi,
                 preferred_element_type=jnp.float32) + bv_ref[0:1]
    vo = jnp.dot(q, wvo_ref[...], precision=hi,
                 preferred_element_type=jnp.float32) + bv_ref[1:2]
    val_ref[0] = _round_bf16_bits(ve) | (_round_bf16_bits(vo) << 16)

    offx = jnp.dot(q, wsx_ref[...], precision=hi,
                   preferred_element_type=jnp.float32) + bsox_ref[...]
    offy = jnp.dot(q, wsy_ref[...], precision=hi,
                   preferred_element_type=jnp.float32) + bsoy_ref[...]

    logits = jnp.dot(q, wa_ref[...], precision=hi,
                     preferred_element_type=jnp.float32) + ba_ref[...]
    m = jnp.max(logits, axis=-1, keepdims=True)
    e = jnp.exp(logits - m)
    denom = jnp.dot(e, ssum_ref[...], precision=hi,
                    preferred_element_type=jnp.float32)
    aw128 = jnp.dot(e / denom, eexp_ref[...], precision=hi,
                    preferred_element_type=jnp.float32)

    lane = lax.broadcasted_iota(jnp.int32, (T, LANES), 1)
    h = lane >> 4
    tx = (lane & 1).astype(jnp.float32)
    ty = ((lane >> 1) & 1).astype(jnp.float32)

    gx = refx_ref[...] + offx
    gy = refy_ref[...] + offy
    x0 = jnp.floor(gx)
    y0 = jnp.floor(gy)
    fx = gx - x0
    fy = gy - y0
    xi = x0 + tx
    yi = y0 + ty
    wx = jnp.where(tx > 0.5, fx, 1.0 - fx)
    wy = jnp.where(ty > 0.5, fy, 1.0 - fy)
    valid = ((xi >= 0.0) & (xi <= WW - 1) & (yi >= 0.0) & (yi <= HH - 1))
    xc = jnp.clip(xi, 0.0, WW - 1).astype(jnp.int32)
    yc = jnp.clip(yi, 0.0, HH - 1).astype(jnp.int32)
    qsrc = yc * WW + xc
    row = b * nq + qsrc
    idx_ref[0] = (row << 3) + h
    wgt_ref[0] = jnp.where(valid, aw128 * wx * wy, 0.0)


def _k1(query, refx, refy, Wve, Wvo, bv2, Wsx, bsox, Wsy, bsoy, Wa, ba,
        Ssum, Eexp):
    B, Nq, D = query.shape
    grid = (B, Nq // T)
    full = lambda shape: pl.BlockSpec(shape, lambda b, j: (0,) * len(shape))
    return pl.pallas_call(
        functools.partial(_k1_body, nq=Nq),
        grid=grid,
        in_specs=[
            pl.BlockSpec((1, T, D), lambda b, j: (b, j, 0)),
            pl.BlockSpec((T, 1), lambda b, j: (j, 0)),
            pl.BlockSpec((T, 1), lambda b, j: (j, 0)),
            full((D, LANES)), full((D, LANES)), full((2, LANES)),
            full((D, LANES)), full((1, LANES)),
            full((D, LANES)), full((1, LANES)),
            full((D, NH * NP)), full((1, NH * NP)),
            full((NH * NP, NH * NP)), full((NH * NP, LANES)),
        ],
        out_specs=[
            pl.BlockSpec((1, T, LANES), lambda b, j: (b, j, 0)),
            pl.BlockSpec((1, T, LANES), lambda b, j: (b, j, 0)),
            pl.BlockSpec((1, T, LANES), lambda b, j: (b, j, 0)),
        ],
        out_shape=[
            jax.ShapeDtypeStruct((B, Nq, LANES), jnp.int32),
            jax.ShapeDtypeStruct((B, Nq, LANES), jnp.int32),
            jax.ShapeDtypeStruct((B, Nq, LANES), jnp.float32),
        ],
    )(query, refx, refy, Wve, Wvo, bv2, Wsx, bsox, Wsy, bsoy, Wa, ba,
      Ssum, Eexp)


def _k2_body(table, idxh, wgth, outh, idx_v, wgt_v, rows_v, out_v,
             sem_r0, sem_r1, sem_i0, sem_i1, sem_w0, sem_w1, sem_o0, sem_o1,
             *, npw, nchunk):
    wid = lax.axis_index("s") * NC + lax.axis_index("c")
    sem_r = (sem_r0, sem_r1)
    sem_i = (sem_i0, sem_i1)
    sem_w = (sem_w0, sem_w1)
    sem_o = (sem_o0, sem_o1)
    NIR = (G * TAPS) // 128

    def ibase(c):
        return pl.multiple_of(wid * npw + c * G, G)

    def ebase(c):
        return pl.multiple_of(ibase(c) * TAPS, G * TAPS)

    def rbase(c):
        return pl.multiple_of(ebase(c) // 128, NIR)

    def idx_copy(c, buf):
        return pltpu.make_async_copy(idxh.at[pl.ds(rbase(c), NIR)],
                                     idx_v.at[buf], sem_i[buf])

    def wgt_copy(c, buf):
        return pltpu.make_async_copy(wgth.at[pl.ds(ebase(c), G * TAPS)],
                                     wgt_v.at[buf], sem_w[buf])

    def gather_copy(c, buf, j):
        return pltpu.make_async_copy(
            table.at[idx_v.at[buf].at[j]],
            rows_v.at[buf].at[pl.ds(j * 128, 128)], sem_r[buf])

    def out_copy(c, buf):
        return pltpu.make_async_copy(out_v.at[buf],
                                     outh.at[pl.ds(ibase(c), G)], sem_o[buf])

    idx_copy(0, 0).start()
    wgt_copy(0, 0).start()
    idx_copy(1, 1).start()
    wgt_copy(1, 1).start()
    idx_copy(0, 0).wait()
    for j in range(NIR):
        gather_copy(0, 0, j).start()

    def pair_body(i, carry):
        for b in (0, 1):
            c = i * 2 + b
            nb = 1 - b
            for j in range(NIR):
                gather_copy(c, b, j).wait()
            wgt_copy(c, b).wait()

            @pl.when(c + 1 < nchunk)
            def _():
                idx_copy(c + 1, nb).wait()
                for j in range(NIR):
                    gather_copy(c + 1, nb, j).start()

            @pl.when(c >= 2)
            def _():
                out_copy(c - 2, b).wait()

            def item_body(k, carry2):
                e0 = k * TAPS
                w16 = wgt_v[b, pl.ds(e0, 16)]
                acc0e = jnp.zeros((16,), jnp.float32)
                acc1e = jnp.zeros((16,), jnp.float32)
                acc0o = jnp.zeros((16,), jnp.float32)
                acc1o = jnp.zeros((16,), jnp.float32)
                for t in range(TAPS):
                    wk = jnp.broadcast_to(w16[t], (16,))
                    word = rows_v[b, e0 + t, pl.ds(0, 16)]
                    even = plsc.bitcast(word << 16, jnp.float32)
                    odd = plsc.bitcast(word & jnp.int32(-65536), jnp.float32)
                    if t % 2 == 0:
                        acc0e = acc0e + wk * even
                        acc0o = acc0o + wk * odd
                    else:
                        acc1e = acc1e + wk * even
                        acc1o = acc1o + wk * odd
                out_v[b, k, pl.ds(0, 16)] = acc0e + acc1e
                out_v[b, k, pl.ds(16, 16)] = acc0o + acc1o
                return carry2

            lax.fori_loop(0, G, item_body, 0, unroll=False)
            out_copy(c, b).start()

            @pl.when(c + 2 < nchunk)
            def _():
                idx_copy(c + 2, b).start()
                wgt_copy(c + 2, b).start()
        return carry

    lax.fori_loop(0, nchunk // 2, pair_body, 0, unroll=False)
    out_copy(nchunk - 2, 0).wait()
    out_copy(nchunk - 1, 1).wait()


def _k2(table, idx2d, wgt_flat, nitems):
    npw = nitems // NW
    nchunk = npw // G
    mesh = plsc.VectorSubcoreMesh(core_axis_name="c", subcore_axis_name="s",
                                  num_cores=NC, num_subcores=NS)
    kern = functools.partial(
        pl.kernel,
        mesh=mesh,
        out_type=jax.ShapeDtypeStruct((nitems, DH), jnp.float32),
        scratch_types=[
            pltpu.VMEM((2, (G * TAPS) // 128, 128), jnp.int32),
            pltpu.VMEM((2, G * TAPS), jnp.float32),
            pltpu.VMEM((2, G * TAPS, DH // 2), jnp.int32),
            pltpu.VMEM((2, G, DH), jnp.float32),
        ] + [pltpu.SemaphoreType.DMA] * 8,
        compiler_params=pltpu.CompilerParams(needs_layout_passes=False,
                                             use_tc_tiling_on_sc=False),
    )(functools.partial(_k2_body, npw=npw, nchunk=nchunk))
    return kern(table, idx2d, wgt_flat)


def _k3_body(s_ref, q_ref, wo_ref, bo_ref, out_ref):
    out_ref[0] = (jnp.dot(s_ref[0].astype(jnp.bfloat16), wo_ref[...],
                          preferred_element_type=jnp.float32)
                  + bo_ref[...] + 2.0 * q_ref[0])


def _k3(sampled, query, Wo, bo):
    B, Nq, D = query.shape
    grid = (B, Nq // T)
    return pl.pallas_call(
        _k3_body,
        grid=grid,
        in_specs=[
            pl.BlockSpec((1, T, D), lambda b, j: (b, j, 0)),
            pl.BlockSpec((1, T, D), lambda b, j: (b, j, 0)),
            pl.BlockSpec((D, D), lambda b, j: (0, 0)),
            pl.BlockSpec((1, D), lambda b, j: (0, 0)),
        ],
        out_specs=pl.BlockSpec((1, T, D), lambda b, j: (b, j, 0)),
        out_shape=jax.ShapeDtypeStruct((B, Nq, D), jnp.float32),
    )(sampled, query, Wo, bo)


def kernel(query, ref_points, Wv, bv, Ws, bso, Wa, ba, Wo, bo):
    B, Nq, D = query.shape
    nitems = B * Nq * NH

    bf16 = jnp.bfloat16
    Wsx = jnp.repeat(Ws[:, 0::2], 4, axis=1).astype(bf16)
    Wsy = jnp.repeat(Ws[:, 1::2], 4, axis=1).astype(bf16)
    bsox = jnp.repeat(bso[0::2], 4)[None, :]
    bsoy = jnp.repeat(bso[1::2], 4)[None, :]
    Wve = Wv[:, 0::2].astype(bf16)
    Wvo = Wv[:, 1::2].astype(bf16)
    bv2 = jnp.stack([bv[0::2], bv[1::2]])
    eye32 = jnp.eye(NH * NP, dtype=jnp.float32)
    Eexp = jnp.repeat(eye32, 4, axis=1)
    Ssum = jnp.repeat(jnp.repeat(jnp.eye(NH, dtype=jnp.float32), NP, axis=0),
                      NP, axis=1)
    refx = (ref_points[:, 0, 0] * WW - 0.5).reshape(Nq, 1)
    refy = (ref_points[:, 0, 1] * HH - 0.5).reshape(Nq, 1)
    ch = jnp.arange(D)
    hh, rr = ch // DH, ch % DH
    perm = hh * DH + jnp.where(rr < DH // 2, 2 * rr, 2 * (rr - DH // 2) + 1)
    Wo_perm = Wo[perm, :].astype(bf16)

    value, idx, wgt = _k1(query, refx, refy, Wve, Wvo, bv2,
                          Wsx, bsox, Wsy, bsoy, Wa.astype(bf16),
                          ba.reshape(1, -1), Ssum, Eexp)

    table = value.reshape(B * Nq * NH, DH // 2)
    idx2d = idx.reshape((nitems * TAPS) // 128, 128)
    wgt_flat = wgt.reshape(nitems * TAPS)
    sc_out = _k2(table, idx2d, wgt_flat, nitems)

    sampled = sc_out.reshape(B, Nq, D)
    return _k3(sampled, query, Wo_perm, bo.reshape(1, -1))


# speedup vs baseline: 16.4166x; 1.0458x over previous
import functools

import jax
import jax.numpy as jnp
from jax import lax
from jax.experimental import pallas as pl
from jax.experimental.pallas import tpu as pltpu
from jax.experimental.pallas import tpu_sc as plsc

EMBED = 256
HH = 129
WW = 256
NH = 8
NP = 4
DH = EMBED // NH
LANES = NH * NP * 4
TAPS = NP * 4

T = 256

NC = 2
NS = 16
NW = NC * NS
G = 64


def _round_bf16_bits(v):
    bits = lax.bitcast_convert_type(v, jnp.int32)
    return ((bits + 0x7FFF + ((bits >> 16) & 1)) >> 16) & 0xFFFF


def _k1_body(q_ref, refx_ref, refy_ref, wve_ref, wvo_ref, bv_ref,
             wsx_ref, bsox_ref, wsy_ref, bsoy_ref, wa_ref, ba_ref,
             ssum_ref, eexp_ref, val_ref, idx_ref, wgt_ref, *, nq):
    b = pl.program_id(0)
    qf = q_ref[0]
    q = qf.astype(jnp.bfloat16)
    hi = None

    ve = jnp.dot(q, wve_ref[...], precision=hi,
                 preferred_element_type=jnp.float32) + bv_ref[0:1]
    vo = jnp.dot(q, wvo_ref[...], precision=hi,
                 preferred_element_type=jnp.float32) + bv_ref[1:2]
    val_ref[0] = _round_bf16_bits(ve) | (_round_bf16_bits(vo) << 16)

    offx = jnp.dot(q, wsx_ref[...], precision=hi,
                   preferred_element_type=jnp.float32) + bsox_ref[...]
    offy = jnp.dot(q, wsy_ref[...], precision=hi,
                   preferred_element_type=jnp.float32) + bsoy_ref[...]

    logits = jnp.dot(q, wa_ref[...], precision=hi,
                     preferred_element_type=jnp.float32) + ba_ref[...]
    m = jnp.max(logits, axis=-1, keepdims=True)
    e = jnp.exp(logits - m)
    denom = jnp.dot(e, ssum_ref[...], precision=hi,
                    preferred_element_type=jnp.float32)
    aw128 = jnp.dot(e / denom, eexp_ref[...], precision=hi,
                    preferred_element_type=jnp.float32)

    lane = lax.broadcasted_iota(jnp.int32, (T, LANES), 1)
    h = lane >> 4
    tx = (lane & 1).astype(jnp.float32)
    ty = ((lane >> 1) & 1).astype(jnp.float32)

    gx = refx_ref[...] + offx
    gy = refy_ref[...] + offy
    x0 = jnp.floor(gx)
    y0 = jnp.floor(gy)
    fx = gx - x0
    fy = gy - y0
    xi = x0 + tx
    yi = y0 + ty
    wx = jnp.where(tx > 0.5, fx, 1.0 - fx)
    wy = jnp.where(ty > 0.5, fy, 1.0 - fy)
    valid = ((xi >= 0.0) & (xi <= WW - 1) & (yi >= 0.0) & (yi <= HH - 1))
    xc = jnp.clip(xi, 0.0, WW - 1).astype(jnp.int32)
    yc = jnp.clip(yi, 0.0, HH - 1).astype(jnp.int32)
    qsrc = yc * WW + xc
    row = b * nq + qsrc
    idx_ref[0] = (row << 3) + h
    wb = _round_bf16_bits(jnp.where(valid, aw128 * wx * wy, 0.0))
    wgt_ref[0] = wb | (wb << 16)


def _k1(query, refx, refy, Wve, Wvo, bv2, Wsx, bsox, Wsy, bsoy, Wa, ba,
        Ssum, Eexp):
    B, Nq, D = query.shape
    grid = (B, Nq // T)
    full = lambda shape: pl.BlockSpec(shape, lambda b, j: (0,) * len(shape))
    return pl.pallas_call(
        functools.partial(_k1_body, nq=Nq),
        grid=grid,
        in_specs=[
            pl.BlockSpec((1, T, D), lambda b, j: (b, j, 0)),
            pl.BlockSpec((T, 1), lambda b, j: (j, 0)),
            pl.BlockSpec((T, 1), lambda b, j: (j, 0)),
            full((D, LANES)), full((D, LANES)), full((2, LANES)),
            full((D, LANES)), full((1, LANES)),
            full((D, LANES)), full((1, LANES)),
            full((D, NH * NP)), full((1, NH * NP)),
            full((NH * NP, NH * NP)), full((NH * NP, LANES)),
        ],
        out_specs=[
            pl.BlockSpec((1, T, LANES), lambda b, j: (b, j, 0)),
            pl.BlockSpec((1, T, LANES), lambda b, j: (b, j, 0)),
            pl.BlockSpec((1, T, LANES), lambda b, j: (b, j, 0)),
        ],
        out_shape=[
            jax.ShapeDtypeStruct((B, Nq, LANES), jnp.int32),
            jax.ShapeDtypeStruct((B, Nq, LANES), jnp.int32),
            jax.ShapeDtypeStruct((B, Nq, LANES), jnp.int32),
        ],
    )(query, refx, refy, Wve, Wvo, bv2, Wsx, bsox, Wsy, bsoy, Wa, ba,
      Ssum, Eexp)


def _k2_body(table, idxh, wgth, outh, idx_v, wgt_v, rows_v, out_v,
             sem_r0, sem_r1, sem_i0, sem_i1, sem_w0, sem_w1, sem_o0, sem_o1,
             *, npw, nchunk):
    wid = lax.axis_index("s") * NC + lax.axis_index("c")
    sem_r = (sem_r0, sem_r1)
    sem_i = (sem_i0, sem_i1)
    sem_w = (sem_w0, sem_w1)
    sem_o = (sem_o0, sem_o1)
    NIR = (G * TAPS) // 128

    def ibase(c):
        return pl.multiple_of(wid * npw + c * G, G)

    def ebase(c):
        return pl.multiple_of(ibase(c) * TAPS, G * TAPS)

    def rbase(c):
        return pl.multiple_of(ebase(c) // 128, NIR)

    def idx_copy(c, buf):
        return pltpu.make_async_copy(idxh.at[pl.ds(rbase(c), NIR)],
                                     idx_v.at[buf], sem_i[buf])

    def wgt_copy(c, buf):
        return pltpu.make_async_copy(wgth.at[pl.ds(ebase(c), G * TAPS)],
                                     wgt_v.at[buf], sem_w[buf])

    def gather_copy(c, buf, j):
        return pltpu.make_async_copy(
            table.at[idx_v.at[buf].at[j]],
            rows_v.at[buf].at[pl.ds(j * 128, 128)], sem_r[buf])

    def out_copy(c, buf):
        return pltpu.make_async_copy(out_v.at[buf],
                                     outh.at[pl.ds(ibase(c), G)], sem_o[buf])

    idx_copy(0, 0).start()
    wgt_copy(0, 0).start()
    idx_copy(1, 1).start()
    wgt_copy(1, 1).start()
    idx_copy(0, 0).wait()
    for j in range(NIR):
        gather_copy(0, 0, j).start()

    def pair_body(i, carry):
        for b in (0, 1):
            c = i * 2 + b
            nb = 1 - b
            for j in range(NIR):
                gather_copy(c, b, j).wait()
            wgt_copy(c, b).wait()

            @pl.when(c + 1 < nchunk)
            def _():
                idx_copy(c + 1, nb).wait()
                for j in range(NIR):
                    gather_copy(c + 1, nb, j).start()

            @pl.when(c >= 2)
            def _():
                out_copy(c - 2, b).wait()

            def item_body(k, carry2):
                e0 = k * TAPS
                w16 = wgt_v[b, pl.ds(e0, 16)]
                acc0 = jnp.zeros((32,), jnp.bfloat16)
                acc1 = jnp.zeros((32,), jnp.bfloat16)
                for t in range(TAPS):
                    wk = plsc.bitcast(jnp.broadcast_to(w16[t], (16,)),
                                      jnp.bfloat16)
                    vals = plsc.bitcast(rows_v[b, e0 + t, pl.ds(0, 16)],
                                        jnp.bfloat16)
                    if t % 2 == 0:
                        acc0 = acc0 + wk * vals
                    else:
                        acc1 = acc1 + wk * vals
                out_v[b, k, pl.ds(0, 32)] = acc0 + acc1
                return carry2

            lax.fori_loop(0, G, item_body, 0, unroll=False)
            out_copy(c, b).start()

            @pl.when(c + 2 < nchunk)
            def _():
                idx_copy(c + 2, b).start()
                wgt_copy(c + 2, b).start()
        return carry

    lax.fori_loop(0, nchunk // 2, pair_body, 0, unroll=False)
    out_copy(nchunk - 2, 0).wait()
    out_copy(nchunk - 1, 1).wait()


def _k2(table, idx2d, wgt_flat, nitems):
    npw = nitems // NW
    nchunk = npw // G
    mesh = plsc.VectorSubcoreMesh(core_axis_name="c", subcore_axis_name="s",
                                  num_cores=NC, num_subcores=NS)
    kern = functools.partial(
        pl.kernel,
        mesh=mesh,
        out_type=jax.ShapeDtypeStruct((nitems, DH), jnp.bfloat16),
        scratch_types=[
            pltpu.VMEM((2, (G * TAPS) // 128, 128), jnp.int32),
            pltpu.VMEM((2, G * TAPS), jnp.int32),
            pltpu.VMEM((2, G * TAPS, DH // 2), jnp.int32),
            pltpu.VMEM((2, G, DH), jnp.bfloat16),
        ] + [pltpu.SemaphoreType.DMA] * 8,
        compiler_params=pltpu.CompilerParams(needs_layout_passes=False,
                                             use_tc_tiling_on_sc=False),
    )(functools.partial(_k2_body, npw=npw, nchunk=nchunk))
    return kern(table, idx2d, wgt_flat)


def _k3_body(s_ref, q_ref, wo_ref, bo_ref, out_ref):
    out_ref[0] = (jnp.dot(s_ref[0], wo_ref[...],
                          preferred_element_type=jnp.float32)
                  + bo_ref[...] + 2.0 * q_ref[0])


def _k3(sampled, query, Wo, bo):
    B, Nq, D = query.shape
    grid = (B, Nq // T)
    return pl.pallas_call(
        _k3_body,
        grid=grid,
        in_specs=[
            pl.BlockSpec((1, T, D), lambda b, j: (b, j, 0)),
            pl.BlockSpec((1, T, D), lambda b, j: (b, j, 0)),
            pl.BlockSpec((D, D), lambda b, j: (0, 0)),
            pl.BlockSpec((1, D), lambda b, j: (0, 0)),
        ],
        out_specs=pl.BlockSpec((1, T, D), lambda b, j: (b, j, 0)),
        out_shape=jax.ShapeDtypeStruct((B, Nq, D), jnp.float32),
    )(sampled, query, Wo, bo)


def kernel(query, ref_points, Wv, bv, Ws, bso, Wa, ba, Wo, bo):
    B, Nq, D = query.shape
    nitems = B * Nq * NH

    bf16 = jnp.bfloat16
    Wsx = jnp.repeat(Ws[:, 0::2], 4, axis=1).astype(bf16)
    Wsy = jnp.repeat(Ws[:, 1::2], 4, axis=1).astype(bf16)
    bsox = jnp.repeat(bso[0::2], 4)[None, :]
    bsoy = jnp.repeat(bso[1::2], 4)[None, :]
    Wve = Wv[:, 0::2].astype(bf16)
    Wvo = Wv[:, 1::2].astype(bf16)
    bv2 = jnp.stack([bv[0::2], bv[1::2]])
    eye32 = jnp.eye(NH * NP, dtype=jnp.float32)
    Eexp = jnp.repeat(eye32, 4, axis=1)
    Ssum = jnp.repeat(jnp.repeat(jnp.eye(NH, dtype=jnp.float32), NP, axis=0),
                      NP, axis=1)
    refx = (ref_points[:, 0, 0] * WW - 0.5).reshape(Nq, 1)
    refy = (ref_points[:, 0, 1] * HH - 0.5).reshape(Nq, 1)
    value, idx, wgt = _k1(query, refx, refy, Wve, Wvo, bv2,
                          Wsx, bsox, Wsy, bsoy, Wa.astype(bf16),
                          ba.reshape(1, -1), Ssum, Eexp)

    table = value.reshape(B * Nq * NH, DH // 2)
    idx2d = idx.reshape((nitems * TAPS) // 128, 128)
    wgt_flat = wgt.reshape(nitems * TAPS)
    sc_out = _k2(table, idx2d, wgt_flat, nitems)

    sampled = sc_out.reshape(B, Nq, D)
    return _k3(sampled, query, Wo.astype(bf16), bo.reshape(1, -1))


# speedup vs baseline: 21.4232x; 1.3050x over previous
import functools

import jax
import jax.numpy as jnp
from jax import lax
from jax.experimental import pallas as pl
from jax.experimental.pallas import tpu as pltpu
from jax.experimental.pallas import tpu_sc as plsc

EMBED = 256
HH = 129
WW = 256
NH = 8
NP = 4
DH = EMBED // NH
LANES = NH * NP * 4
TAPS = NP * 4

T = 768

NC = 2
NS = 16
NW = NC * NS
G = 96


def _round_bf16_bits(v):
    bits = lax.bitcast_convert_type(v, jnp.int32)
    return ((bits + 0x7FFF + ((bits >> 16) & 1)) >> 16) & 0xFFFF


def _k1_body(q_ref, refx_ref, refy_ref, wve_ref, wvo_ref, bv_ref,
             wsx_ref, bsox_ref, wsy_ref, bsoy_ref, wa_ref, ba_ref,
             ssum_ref, eexp_ref, val_ref, idx_ref, wgt_ref, *, nq):
    b = pl.program_id(0)
    qf = q_ref[0]
    q = qf.astype(jnp.bfloat16)
    hi = None

    ve = jnp.dot(q, wve_ref[...], precision=hi,
                 preferred_element_type=jnp.float32) + bv_ref[0:1]
    vo = jnp.dot(q, wvo_ref[...], precision=hi,
                 preferred_element_type=jnp.float32) + bv_ref[1:2]
    val_ref[0] = _round_bf16_bits(ve) | (_round_bf16_bits(vo) << 16)

    offx = jnp.dot(q, wsx_ref[...], precision=hi,
                   preferred_element_type=jnp.float32) + bsox_ref[...]
    offy = jnp.dot(q, wsy_ref[...], precision=hi,
                   preferred_element_type=jnp.float32) + bsoy_ref[...]

    logits = jnp.dot(q, wa_ref[...], precision=hi,
                     preferred_element_type=jnp.float32) + ba_ref[...]
    m = jnp.max(logits, axis=-1, keepdims=True)
    e = jnp.exp(logits - m)
    denom = jnp.dot(e, ssum_ref[...], precision=hi,
                    preferred_element_type=jnp.float32)
    aw128 = jnp.dot(e / denom, eexp_ref[...], precision=hi,
                    preferred_element_type=jnp.float32)

    lane = lax.broadcasted_iota(jnp.int32, (T, LANES), 1)
    h = lane >> 4
    tx = (lane & 1).astype(jnp.float32)
    ty = ((lane >> 1) & 1).astype(jnp.float32)

    gx = refx_ref[...] + offx
    gy = refy_ref[...] + offy
    x0 = jnp.floor(gx)
    y0 = jnp.floor(gy)
    fx = gx - x0
    fy = gy - y0
    xi = x0 + tx
    yi = y0 + ty
    wx = jnp.where(tx > 0.5, fx, 1.0 - fx)
    wy = jnp.where(ty > 0.5, fy, 1.0 - fy)
    valid = ((xi >= 0.0) & (xi <= WW - 1) & (yi >= 0.0) & (yi <= HH - 1))
    xc = jnp.clip(xi, 0.0, WW - 1).astype(jnp.int32)
    yc = jnp.clip(yi, 0.0, HH - 1).astype(jnp.int32)
    qsrc = yc * WW + xc
    row = b * nq + qsrc
    idx_ref[0] = (row << 3) + h
    wb = _round_bf16_bits(jnp.where(valid, aw128 * wx * wy, 0.0))
    wgt_ref[0] = wb | (wb << 16)


def _k1(query, refx, refy, Wve, Wvo, bv2, Wsx, bsox, Wsy, bsoy, Wa, ba,
        Ssum, Eexp):
    B, Nq, D = query.shape
    grid = (B, Nq // T)
    full = lambda shape: pl.BlockSpec(shape, lambda b, j: (0,) * len(shape))
    return pl.pallas_call(
        functools.partial(_k1_body, nq=Nq),
        grid=grid,
        in_specs=[
            pl.BlockSpec((1, T, D), lambda b, j: (b, j, 0)),
            pl.BlockSpec((T, 1), lambda b, j: (j, 0)),
            pl.BlockSpec((T, 1), lambda b, j: (j, 0)),
            full((D, LANES)), full((D, LANES)), full((2, LANES)),
            full((D, LANES)), full((1, LANES)),
            full((D, LANES)), full((1, LANES)),
            full((D, NH * NP)), full((1, NH * NP)),
            full((NH * NP, NH * NP)), full((NH * NP, LANES)),
        ],
        out_specs=[
            pl.BlockSpec((1, T, LANES), lambda b, j: (b, j, 0)),
            pl.BlockSpec((1, T, LANES), lambda b, j: (b, j, 0)),
            pl.BlockSpec((1, T, LANES), lambda b, j: (b, j, 0)),
        ],
        out_shape=[
            jax.ShapeDtypeStruct((B, Nq, LANES), jnp.int32),
            jax.ShapeDtypeStruct((B, Nq, LANES), jnp.int32),
            jax.ShapeDtypeStruct((B, Nq, LANES), jnp.int32),
        ],
    )(query, refx, refy, Wve, Wvo, bv2, Wsx, bsox, Wsy, bsoy, Wa, ba,
      Ssum, Eexp)


def _k2_body(table, idxh, wgth, outh, idx_v, wgt_v, rows_v, out_v,
             sem_r0, sem_r1, sem_i0, sem_i1, sem_w0, sem_w1, sem_o0, sem_o1,
             *, npw, nchunk):
    wid = lax.axis_index("s") * NC + lax.axis_index("c")
    sem_r = (sem_r0, sem_r1)
    sem_i = (sem_i0, sem_i1)
    sem_w = (sem_w0, sem_w1)
    sem_o = (sem_o0, sem_o1)
    NIR = (G * TAPS) // 128

    def ibase(c):
        return pl.multiple_of(wid * npw + c * G, G)

    def ebase(c):
        return pl.multiple_of(ibase(c) * TAPS, G * TAPS)

    def rbase(c):
        return pl.multiple_of(ebase(c) // 128, NIR)

    def idx_copy(c, buf):
        return pltpu.make_async_copy(idxh.at[pl.ds(rbase(c), NIR)],
                                     idx_v.at[buf], sem_i[buf])

    def wgt_copy(c, buf):
        return pltpu.make_async_copy(wgth.at[pl.ds(ebase(c), G * TAPS)],
                                     wgt_v.at[buf], sem_w[buf])

    def gather_copy(c, buf, j):
        return pltpu.make_async_copy(
            table.at[idx_v.at[buf].at[j]],
            rows_v.at[buf].at[pl.ds(j * 128, 128)], sem_r[buf])

    def out_copy(c, buf):
        return pltpu.make_async_copy(out_v.at[buf],
                                     outh.at[pl.ds(ibase(c), G)], sem_o[buf])

    idx_copy(0, 0).start()
    wgt_copy(0, 0).start()
    idx_copy(1, 1).start()
    wgt_copy(1, 1).start()
    idx_copy(0, 0).wait()
    for j in range(NIR):
        gather_copy(0, 0, j).start()

    def pair_body(i, carry):
        for b in (0, 1):
            c = i * 2 + b
            nb = 1 - b
            for j in range(NIR):
                gather_copy(c, b, j).wait()
            wgt_copy(c, b).wait()

            @pl.when(c + 1 < nchunk)
            def _():
                idx_copy(c + 1, nb).wait()
                for j in range(NIR):
                    gather_copy(c + 1, nb, j).start()

            @pl.when(c >= 2)
            def _():
                out_copy(c - 2, b).wait()

            def item_body(k, carry2):
                e0 = k * TAPS
                w16 = wgt_v[b, pl.ds(e0, 16)]
                acc0 = jnp.zeros((32,), jnp.bfloat16)
                acc1 = jnp.zeros((32,), jnp.bfloat16)
                for t in range(TAPS):
                    wk = plsc.bitcast(jnp.broadcast_to(w16[t], (16,)),
                                      jnp.bfloat16)
                    vals = plsc.bitcast(rows_v[b, e0 + t, pl.ds(0, 16)],
                                        jnp.bfloat16)
                    if t % 2 == 0:
                        acc0 = acc0 + wk * vals
                    else:
                        acc1 = acc1 + wk * vals
                out_v[b, k, pl.ds(0, 32)] = acc0 + acc1
                return carry2

            lax.fori_loop(0, G, item_body, 0, unroll=False)
            out_copy(c, b).start()

            @pl.when(c + 2 < nchunk)
            def _():
                idx_copy(c + 2, b).start()
                wgt_copy(c + 2, b).start()
        return carry

    lax.fori_loop(0, nchunk // 2, pair_body, 0, unroll=False)
    out_copy(nchunk - 2, 0).wait()
    out_copy(nchunk - 1, 1).wait()


def _k2(table, idx2d, wgt_flat, nitems):
    npw = nitems // NW
    nchunk = npw // G
    mesh = plsc.VectorSubcoreMesh(core_axis_name="c", subcore_axis_name="s",
                                  num_cores=NC, num_subcores=NS)
    kern = functools.partial(
        pl.kernel,
        mesh=mesh,
        out_type=jax.ShapeDtypeStruct((nitems, DH), jnp.bfloat16),
        scratch_types=[
            pltpu.VMEM((2, (G * TAPS) // 128, 128), jnp.int32),
            pltpu.VMEM((2, G * TAPS), jnp.int32),
            pltpu.VMEM((2, G * TAPS, DH // 2), jnp.int32),
            pltpu.VMEM((2, G, DH), jnp.bfloat16),
        ] + [pltpu.SemaphoreType.DMA] * 8,
        compiler_params=pltpu.CompilerParams(needs_layout_passes=False,
                                             use_tc_tiling_on_sc=False),
    )(functools.partial(_k2_body, npw=npw, nchunk=nchunk))
    return kern(table, idx2d, wgt_flat)


def _k3_body(s_ref, q_ref, wo_ref, bo_ref, out_ref):
    out_ref[0] = (jnp.dot(s_ref[0], wo_ref[...],
                          preferred_element_type=jnp.float32)
                  + bo_ref[...] + 2.0 * q_ref[0])


def _k3(sampled, query, Wo, bo):
    B, Nq, D = query.shape
    grid = (B, Nq // T)
    return pl.pallas_call(
        _k3_body,
        grid=grid,
        in_specs=[
            pl.BlockSpec((1, T, D), lambda b, j: (b, j, 0)),
            pl.BlockSpec((1, T, D), lambda b, j: (b, j, 0)),
            pl.BlockSpec((D, D), lambda b, j: (0, 0)),
            pl.BlockSpec((1, D), lambda b, j: (0, 0)),
        ],
        out_specs=pl.BlockSpec((1, T, D), lambda b, j: (b, j, 0)),
        out_shape=jax.ShapeDtypeStruct((B, Nq, D), jnp.float32),
    )(sampled, query, Wo, bo)


def kernel(query, ref_points, Wv, bv, Ws, bso, Wa, ba, Wo, bo):
    B, Nq, D = query.shape
    nitems = B * Nq * NH

    bf16 = jnp.bfloat16
    Wsx = jnp.repeat(Ws[:, 0::2], 4, axis=1).astype(bf16)
    Wsy = jnp.repeat(Ws[:, 1::2], 4, axis=1).astype(bf16)
    bsox = jnp.repeat(bso[0::2], 4)[None, :]
    bsoy = jnp.repeat(bso[1::2], 4)[None, :]
    Wve = Wv[:, 0::2].astype(bf16)
    Wvo = Wv[:, 1::2].astype(bf16)
    bv2 = jnp.stack([bv[0::2], bv[1::2]])
    eye32 = jnp.eye(NH * NP, dtype=jnp.float32)
    Eexp = jnp.repeat(eye32, 4, axis=1)
    Ssum = jnp.repeat(jnp.repeat(jnp.eye(NH, dtype=jnp.float32), NP, axis=0),
                      NP, axis=1)
    refx = (ref_points[:, 0, 0] * WW - 0.5).reshape(Nq, 1)
    refy = (ref_points[:, 0, 1] * HH - 0.5).reshape(Nq, 1)
    value, idx, wgt = _k1(query, refx, refy, Wve, Wvo, bv2,
                          Wsx, bsox, Wsy, bsoy, Wa.astype(bf16),
                          ba.reshape(1, -1), Ssum, Eexp)

    table = value.reshape(B * Nq * NH, DH // 2)
    idx2d = idx.reshape((nitems * TAPS) // 128, 128)
    wgt_flat = wgt.reshape(nitems * TAPS)
    sc_out = _k2(table, idx2d, wgt_flat, nitems)

    sampled = sc_out.reshape(B, Nq, D)
    return _k3(sampled, query, Wo.astype(bf16), bo.reshape(1, -1))


# speedup vs baseline: 23.2099x; 1.0834x over previous
import functools

import jax
import jax.numpy as jnp
from jax import lax
from jax.experimental import pallas as pl
from jax.experimental.pallas import tpu as pltpu
from jax.experimental.pallas import tpu_sc as plsc

EMBED = 256
HH = 129
WW = 256
NH = 8
NP = 4
DH = EMBED // NH
LANES = NH * NP * 4
TAPS = NP * 4

T = 768

NC = 2
NS = 16
NW = NC * NS
G = 192


def _round_bf16_bits(v):
    bits = lax.bitcast_convert_type(v, jnp.int32)
    return ((bits + 0x7FFF + ((bits >> 16) & 1)) >> 16) & 0xFFFF


def _k1_body(q_ref, refx_ref, refy_ref, wve_ref, wvo_ref, bv_ref,
             wsx_ref, bsox_ref, wsy_ref, bsoy_ref, wa_ref, ba_ref,
             ssum_ref, eexp_ref, val_ref, idx_ref, wgt_ref, *, nq):
    b = pl.program_id(0)
    qf = q_ref[0]
    q = qf.astype(jnp.bfloat16)
    hi = None

    ve = jnp.dot(q, wve_ref[...], precision=hi,
                 preferred_element_type=jnp.float32) + bv_ref[0:1]
    vo = jnp.dot(q, wvo_ref[...], precision=hi,
                 preferred_element_type=jnp.float32) + bv_ref[1:2]
    val_ref[0] = _round_bf16_bits(ve) | (_round_bf16_bits(vo) << 16)

    offx = jnp.dot(q, wsx_ref[...], precision=hi,
                   preferred_element_type=jnp.float32) + bsox_ref[...]
    offy = jnp.dot(q, wsy_ref[...], precision=hi,
                   preferred_element_type=jnp.float32) + bsoy_ref[...]

    logits = jnp.dot(q, wa_ref[...], precision=hi,
                     preferred_element_type=jnp.float32) + ba_ref[...]
    m = jnp.max(logits, axis=-1, keepdims=True)
    e = jnp.exp(logits - m)
    denom = jnp.dot(e, ssum_ref[...], precision=hi,
                    preferred_element_type=jnp.float32)
    aw128 = jnp.dot(e / denom, eexp_ref[...], precision=hi,
                    preferred_element_type=jnp.float32)

    lane = lax.broadcasted_iota(jnp.int32, (T, LANES), 1)
    h = lane >> 4
    tx = (lane & 1).astype(jnp.float32)
    ty = ((lane >> 1) & 1).astype(jnp.float32)

    gx = refx_ref[...] + offx
    gy = refy_ref[...] + offy
    x0 = jnp.floor(gx)
    y0 = jnp.floor(gy)
    fx = gx - x0
    fy = gy - y0
    xi = x0 + tx
    yi = y0 + ty
    wx = jnp.where(tx > 0.5, fx, 1.0 - fx)
    wy = jnp.where(ty > 0.5, fy, 1.0 - fy)
    valid = ((xi >= 0.0) & (xi <= WW - 1) & (yi >= 0.0) & (yi <= HH - 1))
    xc = jnp.clip(xi, 0.0, WW - 1).astype(jnp.int32)
    yc = jnp.clip(yi, 0.0, HH - 1).astype(jnp.int32)
    qsrc = yc * WW + xc
    row = b * nq + qsrc
    idx_ref[0] = (row << 3) + h
    wb = _round_bf16_bits(jnp.where(valid, aw128 * wx * wy, 0.0))
    wgt_ref[0] = wb | (wb << 16)


def _k1(query, refx, refy, Wve, Wvo, bv2, Wsx, bsox, Wsy, bsoy, Wa, ba,
        Ssum, Eexp):
    B, Nq, D = query.shape
    grid = (B, Nq // T)
    full = lambda shape: pl.BlockSpec(shape, lambda b, j: (0,) * len(shape))
    return pl.pallas_call(
        functools.partial(_k1_body, nq=Nq),
        grid=grid,
        in_specs=[
            pl.BlockSpec((1, T, D), lambda b, j: (b, j, 0)),
            pl.BlockSpec((T, 1), lambda b, j: (j, 0)),
            pl.BlockSpec((T, 1), lambda b, j: (j, 0)),
            full((D, LANES)), full((D, LANES)), full((2, LANES)),
            full((D, LANES)), full((1, LANES)),
            full((D, LANES)), full((1, LANES)),
            full((D, NH * NP)), full((1, NH * NP)),
            full((NH * NP, NH * NP)), full((NH * NP, LANES)),
        ],
        out_specs=[
            pl.BlockSpec((1, T, LANES), lambda b, j: (b, j, 0)),
            pl.BlockSpec((1, T, LANES), lambda b, j: (b, j, 0)),
            pl.BlockSpec((1, T, LANES), lambda b, j: (b, j, 0)),
        ],
        out_shape=[
            jax.ShapeDtypeStruct((B, Nq, LANES), jnp.int32),
            jax.ShapeDtypeStruct((B, Nq, LANES), jnp.int32),
            jax.ShapeDtypeStruct((B, Nq, LANES), jnp.int32),
        ],
    )(query, refx, refy, Wve, Wvo, bv2, Wsx, bsox, Wsy, bsoy, Wa, ba,
      Ssum, Eexp)


def _k2_body(table, idxh, wgth, outh, idx_v, wgt_v, rows_v, out_v,
             sem_r0, sem_r1, sem_i0, sem_i1, sem_w0, sem_w1, sem_o0, sem_o1,
             *, npw, nchunk):
    wid = lax.axis_index("s") * NC + lax.axis_index("c")
    sem_r = (sem_r0, sem_r1)
    sem_i = (sem_i0, sem_i1)
    sem_w = (sem_w0, sem_w1)
    sem_o = (sem_o0, sem_o1)
    NIR = (G * TAPS) // 128

    def ibase(c):
        return pl.multiple_of(wid * npw + c * G, G)

    def ebase(c):
        return pl.multiple_of(ibase(c) * TAPS, G * TAPS)

    def rbase(c):
        return pl.multiple_of(ebase(c) // 128, NIR)

    def idx_copy(c, buf):
        return pltpu.make_async_copy(idxh.at[pl.ds(rbase(c), NIR)],
                                     idx_v.at[buf], sem_i[buf])

    def wgt_copy(c, buf):
        return pltpu.make_async_copy(wgth.at[pl.ds(ebase(c), G * TAPS)],
                                     wgt_v.at[buf], sem_w[buf])

    def gather_copy(c, buf, j):
        return pltpu.make_async_copy(
            table.at[idx_v.at[buf].at[j]],
            rows_v.at[buf].at[pl.ds(j * 128, 128)], sem_r[buf])

    def out_copy(c, buf):
        return pltpu.make_async_copy(out_v.at[buf],
                                     outh.at[pl.ds(ibase(c), G)], sem_o[buf])

    idx_copy(0, 0).start()
    wgt_copy(0, 0).start()
    idx_copy(1, 1).start()
    wgt_copy(1, 1).start()
    idx_copy(0, 0).wait()
    for j in range(NIR):
        gather_copy(0, 0, j).start()

    def pair_body(i, carry):
        for b in (0, 1):
            c = i * 2 + b
            nb = 1 - b
            for j in range(NIR):
                gather_copy(c, b, j).wait()
            wgt_copy(c, b).wait()

            @pl.when(c + 1 < nchunk)
            def _():
                idx_copy(c + 1, nb).wait()
                for j in range(NIR):
                    gather_copy(c + 1, nb, j).start()

            @pl.when(c >= 2)
            def _():
                out_copy(c - 2, b).wait()

            def one_item(k):
                e0 = k * TAPS
                w16 = wgt_v[b, pl.ds(e0, 16)]
                acc0 = jnp.zeros((32,), jnp.bfloat16)
                acc1 = jnp.zeros((32,), jnp.bfloat16)
                for t in range(TAPS):
                    wk = plsc.bitcast(jnp.broadcast_to(w16[t], (16,)),
                                      jnp.bfloat16)
                    vals = plsc.bitcast(rows_v[b, e0 + t, pl.ds(0, 16)],
                                        jnp.bfloat16)
                    if t % 2 == 0:
                        acc0 = acc0 + wk * vals
                    else:
                        acc1 = acc1 + wk * vals
                out_v[b, k, pl.ds(0, 32)] = acc0 + acc1

            def item_body(k2, carry2):
                one_item(k2 * 2)
                one_item(k2 * 2 + 1)
                return carry2

            lax.fori_loop(0, G // 2, item_body, 0, unroll=False)
            out_copy(c, b).start()

            @pl.when(c + 2 < nchunk)
            def _():
                idx_copy(c + 2, b).start()
                wgt_copy(c + 2, b).start()
        return carry

    lax.fori_loop(0, nchunk // 2, pair_body, 0, unroll=False)
    out_copy(nchunk - 2, 0).wait()
    out_copy(nchunk - 1, 1).wait()


def _k2(table, idx2d, wgt_flat, nitems):
    npw = nitems // NW
    nchunk = npw // G
    mesh = plsc.VectorSubcoreMesh(core_axis_name="c", subcore_axis_name="s",
                                  num_cores=NC, num_subcores=NS)
    kern = functools.partial(
        pl.kernel,
        mesh=mesh,
        out_type=jax.ShapeDtypeStruct((nitems, DH), jnp.bfloat16),
        scratch_types=[
            pltpu.VMEM((2, (G * TAPS) // 128, 128), jnp.int32),
            pltpu.VMEM((2, G * TAPS), jnp.int32),
            pltpu.VMEM((2, G * TAPS, DH // 2), jnp.int32),
            pltpu.VMEM((2, G, DH), jnp.bfloat16),
        ] + [pltpu.SemaphoreType.DMA] * 8,
        compiler_params=pltpu.CompilerParams(needs_layout_passes=False,
                                             use_tc_tiling_on_sc=False),
    )(functools.partial(_k2_body, npw=npw, nchunk=nchunk))
    return kern(table, idx2d, wgt_flat)


def _k3_body(s_ref, q_ref, wo_ref, bo_ref, out_ref):
    out_ref[0] = (jnp.dot(s_ref[0], wo_ref[...],
                          preferred_element_type=jnp.float32)
                  + bo_ref[...] + 2.0 * q_ref[0])


def _k3(sampled, query, Wo, bo):
    B, Nq, D = query.shape
    grid = (B, Nq // T)
    return pl.pallas_call(
        _k3_body,
        grid=grid,
        in_specs=[
            pl.BlockSpec((1, T, D), lambda b, j: (b, j, 0)),
            pl.BlockSpec((1, T, D), lambda b, j: (b, j, 0)),
            pl.BlockSpec((D, D), lambda b, j: (0, 0)),
            pl.BlockSpec((1, D), lambda b, j: (0, 0)),
        ],
        out_specs=pl.BlockSpec((1, T, D), lambda b, j: (b, j, 0)),
        out_shape=jax.ShapeDtypeStruct((B, Nq, D), jnp.float32),
    )(sampled, query, Wo, bo)


def kernel(query, ref_points, Wv, bv, Ws, bso, Wa, ba, Wo, bo):
    B, Nq, D = query.shape
    nitems = B * Nq * NH

    bf16 = jnp.bfloat16
    Wsx = jnp.repeat(Ws[:, 0::2], 4, axis=1).astype(bf16)
    Wsy = jnp.repeat(Ws[:, 1::2], 4, axis=1).astype(bf16)
    bsox = jnp.repeat(bso[0::2], 4)[None, :]
    bsoy = jnp.repeat(bso[1::2], 4)[None, :]
    Wve = Wv[:, 0::2].astype(bf16)
    Wvo = Wv[:, 1::2].astype(bf16)
    bv2 = jnp.stack([bv[0::2], bv[1::2]])
    eye32 = jnp.eye(NH * NP, dtype=jnp.float32)
    Eexp = jnp.repeat(eye32, 4, axis=1)
    Ssum = jnp.repeat(jnp.repeat(jnp.eye(NH, dtype=jnp.float32), NP, axis=0),
                      NP, axis=1)
    refx = (ref_points[:, 0, 0] * WW - 0.5).reshape(Nq, 1)
    refy = (ref_points[:, 0, 1] * HH - 0.5).reshape(Nq, 1)
    value, idx, wgt = _k1(query, refx, refy, Wve, Wvo, bv2,
                          Wsx, bsox, Wsy, bsoy, Wa.astype(bf16),
                          ba.reshape(1, -1), Ssum, Eexp)

    table = value.reshape(B * Nq * NH, DH // 2)
    idx2d = idx.reshape((nitems * TAPS) // 128, 128)
    wgt_flat = wgt.reshape(nitems * TAPS)
    sc_out = _k2(table, idx2d, wgt_flat, nitems)

    sampled = sc_out.reshape(B, Nq, D)
    return _k3(sampled, query, Wo.astype(bf16), bo.reshape(1, -1))


# speedup vs baseline: 25.3208x; 1.0909x over previous
import functools

import jax
import jax.numpy as jnp
from jax import lax
from jax.experimental import pallas as pl
from jax.experimental.pallas import tpu as pltpu
from jax.experimental.pallas import tpu_sc as plsc

EMBED = 256
HH = 129
WW = 256
NH = 8
NP = 4
DH = EMBED // NH
LANES = NH * NP * 4
TAPS = NP * 4

T = 2064

NC = 2
NS = 16
NW = NC * NS
G = 192


def _pack_bf16(lo, hi):
    bl = lax.bitcast_convert_type(lo, jnp.int32)
    bh = lax.bitcast_convert_type(hi, jnp.int32)
    rl = ((bl + 0x7FFF + ((bl >> 16) & 1)) >> 16) & 0xFFFF
    rh = (bh + 0x7FFF + ((bh >> 16) & 1)) & jnp.int32(-65536)
    return rl | rh


def _k1_body(q_ref, refx_ref, refy_ref, wve_ref, wvo_ref, bv_ref,
             wsx_ref, bsox_ref, wsy_ref, bsoy_ref, wa_ref, ba_ref,
             ssum_ref, eexp_ref, val_ref, idx_ref, wgt_ref, *, nq):
    b = pl.program_id(0)
    qf = q_ref[0]
    q = qf.astype(jnp.bfloat16)
    hi = None

    ve = jnp.dot(q, wve_ref[...], precision=hi,
                 preferred_element_type=jnp.float32) + bv_ref[0:1]
    vo = jnp.dot(q, wvo_ref[...], precision=hi,
                 preferred_element_type=jnp.float32) + bv_ref[1:2]
    val_ref[0] = _pack_bf16(ve, vo)

    offx = jnp.dot(q, wsx_ref[...], precision=hi,
                   preferred_element_type=jnp.float32) + bsox_ref[...]
    offy = jnp.dot(q, wsy_ref[...], precision=hi,
                   preferred_element_type=jnp.float32) + bsoy_ref[...]

    logits = jnp.dot(q, wa_ref[...], precision=hi,
                     preferred_element_type=jnp.float32) + ba_ref[...]
    m = jnp.max(logits, axis=-1, keepdims=True)
    e = jnp.exp(logits - m)
    denom = jnp.dot(e, ssum_ref[...], precision=hi,
                    preferred_element_type=jnp.float32)
    aw128 = jnp.dot(e / denom, eexp_ref[...], precision=hi,
                    preferred_element_type=jnp.float32)

    lane = lax.broadcasted_iota(jnp.int32, (T, LANES), 1)
    h = lane >> 4
    tx = (lane & 1).astype(jnp.float32)
    ty = ((lane >> 1) & 1).astype(jnp.float32)

    gx = refx_ref[...] + offx
    gy = refy_ref[...] + offy
    x0 = jnp.floor(gx)
    y0 = jnp.floor(gy)
    fx = gx - x0
    fy = gy - y0
    xi = x0 + tx
    yi = y0 + ty
    wx = jnp.where(tx > 0.5, fx, 1.0 - fx)
    wy = jnp.where(ty > 0.5, fy, 1.0 - fy)
    xcf = jnp.clip(xi, 0.0, WW - 1)
    ycf = jnp.clip(yi, 0.0, HH - 1)
    valid = (xi == xcf) & (yi == ycf)
    qsrc = ycf.astype(jnp.int32) * WW + xcf.astype(jnp.int32)
    row = b * nq + qsrc
    idx_ref[0] = (row << 3) + h
    w = jnp.where(valid, aw128 * wx * wy, 0.0)
    wgt_ref[0] = _pack_bf16(w, w)


def _k1(query, refx, refy, Wve, Wvo, bv2, Wsx, bsox, Wsy, bsoy, Wa, ba,
        Ssum, Eexp):
    B, Nq, D = query.shape
    grid = (B, Nq // T)
    full = lambda shape: pl.BlockSpec(shape, lambda b, j: (0,) * len(shape))
    return pl.pallas_call(
        functools.partial(_k1_body, nq=Nq),
        grid=grid,
        in_specs=[
            pl.BlockSpec((1, T, D), lambda b, j: (b, j, 0)),
            pl.BlockSpec((T, 1), lambda b, j: (j, 0)),
            pl.BlockSpec((T, 1), lambda b, j: (j, 0)),
            full((D, LANES)), full((D, LANES)), full((2, LANES)),
            full((D, LANES)), full((1, LANES)),
            full((D, LANES)), full((1, LANES)),
            full((D, NH * NP)), full((1, NH * NP)),
            full((NH * NP, NH * NP)), full((NH * NP, LANES)),
        ],
        out_specs=[
            pl.BlockSpec((1, T, LANES), lambda b, j: (b, j, 0)),
            pl.BlockSpec((1, T, LANES), lambda b, j: (b, j, 0)),
            pl.BlockSpec((1, T, LANES), lambda b, j: (b, j, 0)),
        ],
        out_shape=[
            jax.ShapeDtypeStruct((B, Nq, LANES), jnp.int32),
            jax.ShapeDtypeStruct((B, Nq, LANES), jnp.int32),
            jax.ShapeDtypeStruct((B, Nq, LANES), jnp.int32),
        ],
    )(query, refx, refy, Wve, Wvo, bv2, Wsx, bsox, Wsy, bsoy, Wa, ba,
      Ssum, Eexp)


def _k2_body(table, idxh, wgth, outh, idx_v, wgt_v, rows_v, out_v,
             sem_r0, sem_r1, sem_i0, sem_i1, sem_w0, sem_w1, sem_o0, sem_o1,
             *, npw, nchunk):
    wid = lax.axis_index("s") * NC + lax.axis_index("c")
    sem_r = (sem_r0, sem_r1)
    sem_i = (sem_i0, sem_i1)
    sem_w = (sem_w0, sem_w1)
    sem_o = (sem_o0, sem_o1)
    NIR = (G * TAPS) // 128

    def ibase(c):
        return pl.multiple_of(wid * npw + c * G, G)

    def ebase(c):
        return pl.multiple_of(ibase(c) * TAPS, G * TAPS)

    def rbase(c):
        return pl.multiple_of(ebase(c) // 128, NIR)

    def idx_copy(c, buf):
        return pltpu.make_async_copy(idxh.at[pl.ds(rbase(c), NIR)],
                                     idx_v.at[buf], sem_i[buf])

    def wgt_copy(c, buf):
        return pltpu.make_async_copy(wgth.at[pl.ds(ebase(c), G * TAPS)],
                                     wgt_v.at[buf], sem_w[buf])

    def gather_copy(c, buf, j):
        return pltpu.make_async_copy(
            table.at[idx_v.at[buf].at[j]],
            rows_v.at[buf].at[pl.ds(j * 128, 128)], sem_r[buf])

    def out_copy(c, buf):
        return pltpu.make_async_copy(out_v.at[buf],
                                     outh.at[pl.ds(ibase(c), G)], sem_o[buf])

    idx_copy(0, 0).start()
    wgt_copy(0, 0).start()
    idx_copy(1, 1).start()
    wgt_copy(1, 1).start()
    idx_copy(0, 0).wait()
    for j in range(NIR):
        gather_copy(0, 0, j).start()

    def pair_body(i, carry):
        for b in (0, 1):
            c = i * 2 + b
            nb = 1 - b
            for j in range(NIR):
                gather_copy(c, b, j).wait()
            wgt_copy(c, b).wait()

            @pl.when(c + 1 < nchunk)
            def _():
                idx_copy(c + 1, nb).wait()
                for j in range(NIR):
                    gather_copy(c + 1, nb, j).start()

            @pl.when(c >= 2)
            def _():
                out_copy(c - 2, b).wait()

            def one_item(k):
                e0 = k * TAPS
                w16 = wgt_v[b, pl.ds(e0, 16)]
                acc0 = jnp.zeros((32,), jnp.bfloat16)
                acc1 = jnp.zeros((32,), jnp.bfloat16)
                for t in range(TAPS):
                    wk = plsc.bitcast(jnp.broadcast_to(w16[t], (16,)),
                                      jnp.bfloat16)
                    vals = plsc.bitcast(rows_v[b, e0 + t, pl.ds(0, 16)],
                                        jnp.bfloat16)
                    if t % 2 == 0:
                        acc0 = acc0 + wk * vals
                    else:
                        acc1 = acc1 + wk * vals
                out_v[b, k, pl.ds(0, 32)] = acc0 + acc1

            def item_body(k3, carry2):
                one_item(k3 * 3)
                one_item(k3 * 3 + 1)
                one_item(k3 * 3 + 2)
                return carry2

            lax.fori_loop(0, G // 3, item_body, 0, unroll=False)
            out_copy(c, b).start()

            @pl.when(c + 2 < nchunk)
            def _():
                idx_copy(c + 2, b).start()
                wgt_copy(c + 2, b).start()
        return carry

    lax.fori_loop(0, nchunk // 2, pair_body, 0, unroll=False)
    out_copy(nchunk - 2, 0).wait()
    out_copy(nchunk - 1, 1).wait()


def _k2(table, idx2d, wgt_flat, nitems):
    npw = nitems // NW
    nchunk = npw // G
    mesh = plsc.VectorSubcoreMesh(core_axis_name="c", subcore_axis_name="s",
                                  num_cores=NC, num_subcores=NS)
    kern = functools.partial(
        pl.kernel,
        mesh=mesh,
        out_type=jax.ShapeDtypeStruct((nitems, DH), jnp.bfloat16),
        scratch_types=[
            pltpu.VMEM((2, (G * TAPS) // 128, 128), jnp.int32),
            pltpu.VMEM((2, G * TAPS), jnp.int32),
            pltpu.VMEM((2, G * TAPS, DH // 2), jnp.int32),
            pltpu.VMEM((2, G, DH), jnp.bfloat16),
        ] + [pltpu.SemaphoreType.DMA] * 8,
        compiler_params=pltpu.CompilerParams(needs_layout_passes=False,
                                             use_tc_tiling_on_sc=False),
    )(functools.partial(_k2_body, npw=npw, nchunk=nchunk))
    return kern(table, idx2d, wgt_flat)


def _k3_body(s_ref, q_ref, wo_ref, bo_ref, out_ref):
    out_ref[0] = (jnp.dot(s_ref[0], wo_ref[...],
                          preferred_element_type=jnp.float32)
                  + bo_ref[...] + 2.0 * q_ref[0])


def _k3(sampled, query, Wo, bo):
    B, Nq, D = query.shape
    grid = (B, Nq // T)
    return pl.pallas_call(
        _k3_body,
        grid=grid,
        in_specs=[
            pl.BlockSpec((1, T, D), lambda b, j: (b, j, 0)),
            pl.BlockSpec((1, T, D), lambda b, j: (b, j, 0)),
            pl.BlockSpec((D, D), lambda b, j: (0, 0)),
            pl.BlockSpec((1, D), lambda b, j: (0, 0)),
        ],
        out_specs=pl.BlockSpec((1, T, D), lambda b, j: (b, j, 0)),
        out_shape=jax.ShapeDtypeStruct((B, Nq, D), jnp.float32),
    )(sampled, query, Wo, bo)


def kernel(query, ref_points, Wv, bv, Ws, bso, Wa, ba, Wo, bo):
    B, Nq, D = query.shape
    nitems = B * Nq * NH

    bf16 = jnp.bfloat16
    Wsx = jnp.repeat(Ws[:, 0::2], 4, axis=1).astype(bf16)
    Wsy = jnp.repeat(Ws[:, 1::2], 4, axis=1).astype(bf16)
    bsox = jnp.repeat(bso[0::2], 4)[None, :]
    bsoy = jnp.repeat(bso[1::2], 4)[None, :]
    Wve = Wv[:, 0::2].astype(bf16)
    Wvo = Wv[:, 1::2].astype(bf16)
    bv2 = jnp.stack([bv[0::2], bv[1::2]])
    eye32 = jnp.eye(NH * NP, dtype=jnp.float32)
    Eexp = jnp.repeat(eye32, 4, axis=1)
    Ssum = jnp.repeat(jnp.repeat(jnp.eye(NH, dtype=jnp.float32), NP, axis=0),
                      NP, axis=1)
    refx = (ref_points[:, 0, 0] * WW - 0.5).reshape(Nq, 1)
    refy = (ref_points[:, 0, 1] * HH - 0.5).reshape(Nq, 1)
    value, idx, wgt = _k1(query, refx, refy, Wve, Wvo, bv2,
                          Wsx, bsox, Wsy, bsoy, Wa.astype(bf16),
                          ba.reshape(1, -1), Ssum, Eexp)

    table = value.reshape(B * Nq * NH, DH // 2)
    idx2d = idx.reshape((nitems * TAPS) // 128, 128)
    wgt_flat = wgt.reshape(nitems * TAPS)
    sc_out = _k2(table, idx2d, wgt_flat, nitems)

    sampled = sc_out.reshape(B, Nq, D)
    return _k3(sampled, query, Wo.astype(bf16), bo.reshape(1, -1))
